# baseline (speedup 1.0000x reference)
import functools

import jax
import jax.numpy as jnp
from jax import lax
from jax.experimental import pallas as pl
from jax.experimental.pallas import tpu as pltpu
from jax.experimental.pallas import tpu_sc as plsc

B, C, H, W, K = 64, 8, 128, 128, 128
HW = H * W
N = B * K
NC, NS = 2, 16
NW = NC * NS
IPT = N // NW
NCH = IPT // 16
GPT = C * IPT

_S1, _S2, _S3 = -0.166666546, 8.3321608736e-3, -1.9515295891e-4
_C0, _C1, _C2, _C3 = (-0.499999997251031, 4.16666233237390e-2,
                      -1.38867637746099e-3, 2.43904487962774e-5)
_TWO_OVER_PI = 0.6366197723675814
_PIO2_HI = 1.5707855224609375
_PIO2_LO = 1.0804334124e-05


def _log_1to2(v):
    t = (v - 1.0) / (v + 1.0)
    t2 = t * t
    p = 1.0 + t2 * ((1.0 / 3.0) + t2 * ((1.0 / 5.0) + t2 * ((1.0 / 7.0) + t2 * (1.0 / 9.0))))
    return (2.0 * t) * p


def _sincos(x):
    y = x * _TWO_OVER_PI
    half = jnp.where(y >= 0.0, 0.5, -0.5)
    ki = (y + half).astype(jnp.int32)
    kf = ki.astype(jnp.float32)
    r = (x - kf * _PIO2_HI) - kf * _PIO2_LO
    s = r * r
    sinr = r * (1.0 + s * (_S1 + s * (_S2 + s * _S3)))
    cosr = 1.0 + s * (_C0 + s * (_C1 + s * (_C2 + s * _C3)))
    q1 = (ki & 1) == 0
    q2 = (ki & 2) == 0
    sv = jnp.where(q1, sinr, cosr)
    sv = jnp.where(q2, sv, -sv)
    cv = jnp.where(q1, cosr, -sinr)
    cv = jnp.where(q2, cv, -cv)
    return sv, cv


def _sl1(p, t):
    d = jnp.abs(p - t)
    return jnp.where(d < 1.0, 0.5 * d * d, d - 0.5)


def _sc_body(src_hbm, ind_hbm, mask_hbm, rb_hbm, rr_hbm, out_hbm,
             ind_v, idx_v, rows_v, mask_v, rb_v, rr_v, acc_v, res_v,
             big_v, shared, sem):
    cid = lax.axis_index("c")
    sid = lax.axis_index("s")
    wid = sid * NC + cid
    base = wid * IPT

    pltpu.sync_copy(ind_hbm.at[pl.ds(base, IPT)], ind_v)
    pltpu.sync_copy(mask_hbm.at[pl.ds(base, IPT)], mask_v)
    pltpu.sync_copy(rb_hbm.at[pl.ds(base, IPT)], rb_v.at[pl.ds(0, IPT)])
    pltpu.sync_copy(rb_hbm.at[pl.ds(N + base, IPT)], rb_v.at[pl.ds(IPT, IPT)])
    pltpu.sync_copy(rr_hbm.at[pl.ds(base, IPT)], rr_v.at[pl.ds(0, IPT)])
    pltpu.sync_copy(rr_hbm.at[pl.ds(N + base, IPT)], rr_v.at[pl.ds(IPT, IPT)])

    wbase = (wid * (IPT // K)) * (C * HW)
    for c in range(C):
        for chunk in range(NCH):
            off = (chunk // (K // 16)) * (C * HW) + c * HW
            idx_v[pl.ds((c * NCH + chunk) * 16, 16)] = (
                ind_v[pl.ds(chunk * 16, 16)] + (wbase + off)
            )
    pltpu.async_copy(src_hbm.at[idx_v], rows_v, sem).wait()

    zero = jnp.zeros((16,), jnp.float32)
    s1v = s2v = r1v = r2v = cntv = n1v = n2v = zero
    for j in range(NCH):
        x = [rows_v[pl.ds(c * IPT + j * 16, 16)] for c in range(C)]
        mf = mask_v[pl.ds(j * 16, 16)].astype(jnp.float32)
        tb0 = rb_v[pl.ds(j * 16, 16)]
        tb1 = rb_v[pl.ds(IPT + j * 16, 16)]
        tr0 = rr_v[pl.ds(j * 16, 16)]
        tr1 = rr_v[pl.ds(IPT + j * 16, 16)]

        m1 = jnp.maximum(x[0], x[1])
        lse1 = m1 + _log_1to2(jnp.exp(x[0] - m1) + jnp.exp(x[1] - m1))
        pick1 = jnp.where(tb0 == 1, x[1], x[0]) - lse1
        m2 = jnp.maximum(x[4], x[5])
        lse2 = m2 + _log_1to2(jnp.exp(x[4] - m2) + jnp.exp(x[5] - m2))
        pick2 = jnp.where(tb1 == 1, x[5], x[4]) - lse2
        s1v = s1v + pick1 * mf
        s2v = s2v + pick2 * mf
        cntv = cntv + mf

        w1 = tb0.astype(jnp.float32)
        w2 = tb1.astype(jnp.float32)
        sin0, cos0 = _sincos(tr0)
        sin1, cos1 = _sincos(tr1)
        r1v = r1v + (_sl1(x[2], sin0) + _sl1(x[3], cos0)) * w1
        r2v = r2v + (_sl1(x[6], sin1) + _sl1(x[7], cos1)) * w2
        n1v = n1v + w1
        n2v = n2v + w2

    for i, v in enumerate((s1v, s2v, r1v, r2v, cntv, n1v, n2v, zero)):
        acc_v[pl.ds(i * 16, 16)] = v
    pltpu.sync_copy(acc_v, shared.at[pl.ds(sid * 128, 128)])
    plsc.subcore_barrier()

    @pl.when(sid == 0)
    def _finish():
        pltpu.sync_copy(shared, big_v)
        res = jnp.zeros((16,), jnp.float32)
        lanes = lax.iota(jnp.int32, 16)
        for i in range(7):
            tot = big_v[pl.ds(i * 16, 16)]
            for t in range(1, NS):
                tot = tot + big_v[pl.ds((t * 8 + i) * 16, 16)]
            si = tot[0]
            for l in range(1, 16):
                si = si + tot[l]
            res = jnp.where(lanes == i, si, res)
        res_v[...] = res
        pltpu.sync_copy(res_v, out_hbm.at[cid])


@functools.cache
def _sc_loss():
    return functools.partial(
        pl.kernel,
        out_type=jax.ShapeDtypeStruct((NC, 16), jnp.float32),
        mesh=plsc.VectorSubcoreMesh(core_axis_name="c", subcore_axis_name="s"),
        scratch_types=[
            pltpu.VMEM((IPT,), jnp.int32),
            pltpu.VMEM((GPT,), jnp.int32),
            pltpu.VMEM((GPT,), jnp.float32),
            pltpu.VMEM((IPT,), jnp.int32),
            pltpu.VMEM((2 * IPT,), jnp.int32),
            pltpu.VMEM((2 * IPT,), jnp.float32),
            pltpu.VMEM((8 * 16,), jnp.float32),
            pltpu.VMEM((16,), jnp.float32),
            pltpu.VMEM((NS * 8 * 16,), jnp.float32),
            pltpu.VMEM_SHARED((NS * 8 * 16,), jnp.float32),
            pltpu.SemaphoreType.DMA,
        ],
    )(_sc_body)


def kernel(output, mask, ind, rotbin, rotres):
    src = output.reshape(-1)
    indf = ind.reshape(-1).astype(jnp.int32)
    maskf = mask.reshape(-1).astype(jnp.int32)
    rbf = jnp.transpose(rotbin, (2, 0, 1)).reshape(-1).astype(jnp.int32)
    rrf = jnp.transpose(rotres, (2, 0, 1)).reshape(-1)
    parts = _sc_loss()(src, indf, maskf, rbf, rrf)
    p = parts[0] + parts[1]
    s1, s2, r1, r2 = p[0], p[1], p[2], p[3]
    cnt, n1, n2 = p[4], p[5], p[6]
    zero = jnp.float32(0.0)
    lb1 = jnp.where(cnt > 0, -s1 / cnt, zero)
    lb2 = jnp.where(cnt > 0, -s2 / cnt, zero)
    lr = jnp.where(n1 > 0, r1 / n1, zero) + jnp.where(n2 > 0, r2 / n2, zero)
    return jnp.where(cnt == 0, zero, lb1 + lb2 + lr)


# speedup vs baseline: 1.4149x; 1.4149x over previous
import functools

import jax
import jax.numpy as jnp
from jax import lax
from jax.experimental import pallas as pl
from jax.experimental.pallas import tpu as pltpu
from jax.experimental.pallas import tpu_sc as plsc

B, C, H, W, K = 64, 8, 128, 128, 128
HW = H * W
N = B * K
NC, NS = 2, 16
NW = NC * NS
IPT = N // NW
NCH = IPT // 16
GPT = C * IPT


def _sc_gather_body(src_hbm, ind_hbm, out_hbm, ind_v, idx_v, rows_v, sem):
    wid = lax.axis_index("s") * NC + lax.axis_index("c")
    base = wid * IPT
    pltpu.sync_copy(ind_hbm.at[pl.ds(base, IPT)], ind_v)
    wbase = (wid * (IPT // K)) * (C * HW)
    for chunk in range(NCH):
        iv = ind_v[pl.ds(chunk * 16, 16)]
        boff = wbase + (chunk // (K // 16)) * (C * HW)
        for c in range(C):
            idx_v[pl.ds((c * NCH + chunk) * 16, 16)] = iv + (boff + c * HW)
    pltpu.async_copy(src_hbm.at[idx_v], rows_v, sem).wait()
    pltpu.sync_copy(rows_v, out_hbm.at[wid])


@functools.cache
def _sc_gather():
    return functools.partial(
        pl.kernel,
        out_type=jax.ShapeDtypeStruct((NW, GPT), jnp.float32),
        mesh=plsc.VectorSubcoreMesh(core_axis_name="c", subcore_axis_name="s"),
        scratch_types=[
            pltpu.VMEM((IPT,), jnp.int32),
            pltpu.VMEM((GPT,), jnp.int32),
            pltpu.VMEM((GPT,), jnp.float32),
            pltpu.SemaphoreType.DMA,
        ],
    )(_sc_gather_body)


def _loss_body(pred_ref, mask_ref, tb_ref, tr_ref, out_ref):
    x = [pred_ref[:, c, :] for c in range(C)]
    mf = mask_ref[...].astype(jnp.float32)
    tb0 = tb_ref[0]
    tb1 = tb_ref[1]
    tr0 = tr_ref[0]
    tr1 = tr_ref[1]
    cnt = jnp.sum(mf)

    def pick_logp(a, b, t):
        m = jnp.maximum(a, b)
        lse = m + jnp.log(jnp.exp(a - m) + jnp.exp(b - m))
        return jnp.where(t == 1, b, a) - lse

    s1 = jnp.sum(pick_logp(x[0], x[1], tb0) * mf)
    s2 = jnp.sum(pick_logp(x[4], x[5], tb1) * mf)

    def sl1(p, t):
        d = jnp.abs(p - t)
        return jnp.where(d < 1.0, 0.5 * d * d, d - 0.5)

    w1 = tb0.astype(jnp.float32)
    w2 = tb1.astype(jnp.float32)
    n1 = jnp.sum(w1)
    n2 = jnp.sum(w2)
    r1 = jnp.sum((sl1(x[2], jnp.sin(tr0)) + sl1(x[3], jnp.cos(tr0))) * w1)
    r2 = jnp.sum((sl1(x[6], jnp.sin(tr1)) + sl1(x[7], jnp.cos(tr1))) * w2)

    zero = jnp.float32(0.0)
    lb1 = jnp.where(cnt > 0, -s1 / cnt, zero)
    lb2 = jnp.where(cnt > 0, -s2 / cnt, zero)
    lr = jnp.where(n1 > 0, r1 / n1, zero) + jnp.where(n2 > 0, r2 / n2, zero)
    total = lb1 + lb2 + lr
    out_ref[0, 0] = jnp.where(cnt == 0, zero, total)


_loss = pl.pallas_call(
    _loss_body,
    out_shape=jax.ShapeDtypeStruct((1, 1), jnp.float32),
    out_specs=pl.BlockSpec(memory_space=pltpu.SMEM),
)


def kernel(output, mask, ind, rotbin, rotres):
    src = output.reshape(-1)
    indf = ind.reshape(-1).astype(jnp.int32)
    pred = _sc_gather()(src, indf).reshape(NW, C, IPT)
    tb = jnp.transpose(rotbin, (2, 0, 1)).reshape(2, NW, IPT).astype(jnp.int32)
    tr = jnp.transpose(rotres, (2, 0, 1)).reshape(2, NW, IPT)
    out = _loss(pred, mask.reshape(NW, IPT).astype(jnp.int32), tb, tr)
    return out[0, 0]


# speedup vs baseline: 1.4191x; 1.0030x over previous
import functools

import jax
import jax.numpy as jnp
from jax import lax
from jax.experimental import pallas as pl
from jax.experimental.pallas import tpu as pltpu
from jax.experimental.pallas import tpu_sc as plsc

B, C, H, W, K = 64, 8, 128, 128, 128
HW = H * W
N = B * K
NC, NS = 2, 16
NW = NC * NS
IPT = N // NW
NCH = IPT // 16
GPT = C * IPT


def _sc_gather_body(src_hbm, ind_hbm, out_hbm, ind_v, idx_v, rows_v, sem):
    wid = lax.axis_index("s") * NC + lax.axis_index("c")
    base = wid * IPT
    pltpu.sync_copy(ind_hbm.at[pl.ds(base, IPT)], ind_v)
    wbase = (wid * (IPT // K)) * (C * HW)
    copies = []
    for half in range(4):
        for c in (2 * half, 2 * half + 1):
            for chunk in range(NCH):
                boff = wbase + (chunk // (K // 16)) * (C * HW)
                idx_v[pl.ds((c * NCH + chunk) * 16, 16)] = (
                    ind_v[pl.ds(chunk * 16, 16)] + (boff + c * HW)
                )
        sl = pl.ds(half * 2 * IPT, 2 * IPT)
        copies.append(
            pltpu.async_copy(src_hbm.at[idx_v.at[sl]], rows_v.at[sl], sem)
        )
    for cp in copies:
        cp.wait()
    pltpu.sync_copy(rows_v, out_hbm.at[wid])


@functools.cache
def _sc_gather():
    return functools.partial(
        pl.kernel,
        out_type=jax.ShapeDtypeStruct((NW, GPT), jnp.float32),
        mesh=plsc.VectorSubcoreMesh(core_axis_name="c", subcore_axis_name="s"),
        scratch_types=[
            pltpu.VMEM((IPT,), jnp.int32),
            pltpu.VMEM((GPT,), jnp.int32),
            pltpu.VMEM((GPT,), jnp.float32),
            pltpu.SemaphoreType.DMA,
        ],
    )(_sc_gather_body)


def _loss_body(pred_ref, mask_ref, tb_ref, tr_ref, out_ref):
    x = [pred_ref[:, c, :] for c in range(C)]
    mf = mask_ref[...].astype(jnp.float32)
    tb0 = tb_ref[0]
    tb1 = tb_ref[1]
    tr0 = tr_ref[0]
    tr1 = tr_ref[1]
    cnt = jnp.sum(mf)

    def pick_logp(a, b, t):
        m = jnp.maximum(a, b)
        lse = m + jnp.log(jnp.exp(a - m) + jnp.exp(b - m))
        return jnp.where(t == 1, b, a) - lse

    s1 = jnp.sum(pick_logp(x[0], x[1], tb0) * mf)
    s2 = jnp.sum(pick_logp(x[4], x[5], tb1) * mf)

    def sl1(p, t):
        d = jnp.abs(p - t)
        return jnp.where(d < 1.0, 0.5 * d * d, d - 0.5)

    w1 = tb0.astype(jnp.float32)
    w2 = tb1.astype(jnp.float32)
    n1 = jnp.sum(w1)
    n2 = jnp.sum(w2)
    r1 = jnp.sum((sl1(x[2], jnp.sin(tr0)) + sl1(x[3], jnp.cos(tr0))) * w1)
    r2 = jnp.sum((sl1(x[6], jnp.sin(tr1)) + sl1(x[7], jnp.cos(tr1))) * w2)

    zero = jnp.float32(0.0)
    lb1 = jnp.where(cnt > 0, -s1 / cnt, zero)
    lb2 = jnp.where(cnt > 0, -s2 / cnt, zero)
    lr = jnp.where(n1 > 0, r1 / n1, zero) + jnp.where(n2 > 0, r2 / n2, zero)
    total = lb1 + lb2 + lr
    out_ref[0, 0] = jnp.where(cnt == 0, zero, total)


_loss = pl.pallas_call(
    _loss_body,
    out_shape=jax.ShapeDtypeStruct((1, 1), jnp.float32),
    out_specs=pl.BlockSpec(memory_space=pltpu.SMEM),
)


def kernel(output, mask, ind, rotbin, rotres):
    src = output.reshape(-1)
    indf = ind.reshape(-1).astype(jnp.int32)
    pred = _sc_gather()(src, indf).reshape(NW, C, IPT)
    tb = jnp.transpose(rotbin, (2, 0, 1)).reshape(2, NW, IPT).astype(jnp.int32)
    tr = jnp.transpose(rotres, (2, 0, 1)).reshape(2, NW, IPT)
    out = _loss(pred, mask.reshape(NW, IPT).astype(jnp.int32), tb, tr)
    return out[0, 0]
